# baseline (speedup 1.0000x reference)
import jax
import jax.numpy as jnp
from jax.experimental import pallas as pl

BLOCK_M = 2000


def _linear_block(x_ref, wt_ref, b_ref, o_ref):
    o_ref[...] = (
        jnp.dot(x_ref[...], wt_ref[...], preferred_element_type=jnp.float32)
        + b_ref[...]
    )


def kernel(x, W, b):
    n, k = x.shape
    c = W.shape[0]
    wt = W.T
    b2 = b.reshape(1, c)
    grid = (n // BLOCK_M,)
    return pl.pallas_call(
        _linear_block,
        grid=grid,
        in_specs=[
            pl.BlockSpec((BLOCK_M, k), lambda i: (i, 0)),
            pl.BlockSpec((k, c), lambda i: (0, 0)),
            pl.BlockSpec((1, c), lambda i: (0, 0)),
        ],
        out_specs=pl.BlockSpec((BLOCK_M, c), lambda i: (i, 0)),
        out_shape=jax.ShapeDtypeStruct((n, c), jnp.float32),
    )(x, wt, b2)


# speedup vs baseline: 1.3347x; 1.3347x over previous
import jax
import jax.numpy as jnp
from jax.experimental import pallas as pl

BLOCK_M = 10000


def _linear_block(x_ref, wt_ref, b_ref, o_ref):
    o_ref[...] = (
        jnp.dot(x_ref[...], wt_ref[...], preferred_element_type=jnp.float32)
        + b_ref[...]
    )


def kernel(x, W, b):
    n, k = x.shape
    c = W.shape[0]
    wt = W.T
    b2 = b.reshape(1, c)
    grid = (n // BLOCK_M,)
    return pl.pallas_call(
        _linear_block,
        grid=grid,
        in_specs=[
            pl.BlockSpec((BLOCK_M, k), lambda i: (i, 0)),
            pl.BlockSpec((k, c), lambda i: (0, 0)),
            pl.BlockSpec((1, c), lambda i: (0, 0)),
        ],
        out_specs=pl.BlockSpec((BLOCK_M, c), lambda i: (i, 0)),
        out_shape=jax.ShapeDtypeStruct((n, c), jnp.float32),
    )(x, wt, b2)
